# Initial kernel scaffold; baseline (speedup 1.0000x reference)
import jax
import jax.numpy as jnp
from jax.experimental import pallas as pl


def kernel(x, edge_index, edge_weight, Wf1, bf1, Wfs1, bfs1, Wl1, Wls1, Wf2, bf2, Wfs2, bfs2, Wl2, Wls2, Wo, bo):
    raise NotImplementedError("write your pallas kernel here")


# speedup vs baseline: 1.8981x; 1.8981x over previous
import functools

import jax
import jax.numpy as jnp
from jax import lax
from jax.experimental import pallas as pl
from jax.experimental.pallas import tpu as pltpu
from jax.experimental.pallas import tpu_sc as plsc

N = 10000
NP = 10240
D = 128
E = 320000
NC, NS, L = 2, 16, 16
NW = NC * NS
EPW = 10240
EP = NW * EPW
C = 64
NCHUNK = EPW // C
ROWS_PS = NP // NS
BR = 1280
GRID = NP // BR


def _dense_body(x_ref, wf_ref, bf_ref, wfs_ref, bfs_ref, wl_ref, wls_ref,
                gb_ref, m_ref, skip_ref):
    x = x_ref[...]
    gb_ref[...] = jnp.dot(x, wf_ref[...], preferred_element_type=jnp.float32) + bf_ref[...]
    bgs = jnp.dot(x, wfs_ref[...], preferred_element_type=jnp.float32) + bfs_ref[...]
    m_ref[...] = jnp.dot(x, wl_ref[...], preferred_element_type=jnp.float32)
    pre = bgs[:, D:] * jnp.dot(x, wls_ref[...], preferred_element_type=jnp.float32) + bgs[:, :D]
    skip_ref[...] = jnp.maximum(pre, 0.0)


_dense = pl.pallas_call(
    _dense_body,
    grid=(GRID,),
    in_specs=[
        pl.BlockSpec((BR, D), lambda i: (i, 0)),
        pl.BlockSpec((D, 2 * D), lambda i: (0, 0)),
        pl.BlockSpec((1, 2 * D), lambda i: (0, 0)),
        pl.BlockSpec((D, 2 * D), lambda i: (0, 0)),
        pl.BlockSpec((1, 2 * D), lambda i: (0, 0)),
        pl.BlockSpec((D, D), lambda i: (0, 0)),
        pl.BlockSpec((D, D), lambda i: (0, 0)),
    ],
    out_specs=[
        pl.BlockSpec((BR, 2 * D), lambda i: (i, 0)),
        pl.BlockSpec((BR, D), lambda i: (i, 0)),
        pl.BlockSpec((BR, D), lambda i: (i, 0)),
    ],
    out_shape=[
        jax.ShapeDtypeStruct((NP, 2 * D), jnp.float32),
        jax.ShapeDtypeStruct((NP, D), jnp.float32),
        jax.ShapeDtypeStruct((NP, D), jnp.float32),
    ],
)


def _combine1_body(skip_ref, a0_ref, a1_ref, d0_ref, d1_ref, h_ref):
    deg = d0_ref[...][:, :1] + d1_ref[...][:, :1]
    scale = 1.0 / jnp.maximum(deg, 1.0)
    h = skip_ref[...] + (a0_ref[...] + a1_ref[...]) * scale
    h_ref[...] = jnp.where(h >= 0, h, 0.01 * h)


_combine1 = pl.pallas_call(
    _combine1_body,
    grid=(GRID,),
    in_specs=[
        pl.BlockSpec((BR, D), lambda i: (i, 0)),
        pl.BlockSpec((BR, D), lambda i: (i, 0)),
        pl.BlockSpec((BR, D), lambda i: (GRID + i, 0)),
        pl.BlockSpec((BR, D), lambda i: (i, 0)),
        pl.BlockSpec((BR, D), lambda i: (GRID + i, 0)),
    ],
    out_specs=pl.BlockSpec((BR, D), lambda i: (i, 0)),
    out_shape=jax.ShapeDtypeStruct((NP, D), jnp.float32),
)


def _combine2_body(skip_ref, a0_ref, a1_ref, d0_ref, d1_ref, wo_ref, bo_ref, y_ref):
    deg = d0_ref[...][:, :1] + d1_ref[...][:, :1]
    scale = 1.0 / jnp.maximum(deg, 1.0)
    h = skip_ref[...] + (a0_ref[...] + a1_ref[...]) * scale
    h = jnp.where(h >= 0, h, 0.01 * h)
    y_ref[...] = jnp.dot(h, wo_ref[...], preferred_element_type=jnp.float32) + bo_ref[...]


_combine2 = pl.pallas_call(
    _combine2_body,
    grid=(GRID,),
    in_specs=[
        pl.BlockSpec((BR, D), lambda i: (i, 0)),
        pl.BlockSpec((BR, D), lambda i: (i, 0)),
        pl.BlockSpec((BR, D), lambda i: (GRID + i, 0)),
        pl.BlockSpec((BR, D), lambda i: (i, 0)),
        pl.BlockSpec((BR, D), lambda i: (GRID + i, 0)),
        pl.BlockSpec((D, 1), lambda i: (0, 0)),
        pl.BlockSpec((1, 1), lambda i: (0, 0)),
    ],
    out_specs=pl.BlockSpec((BR, 1), lambda i: (i, 0)),
    out_shape=jax.ShapeDtypeStruct((NP, 1), jnp.float32),
)


SUPER = 16
NSUPER = NCHUNK // SUPER
OL = 64


def _make_sc_edge(with_deg: bool, stage: int = 9):
    mesh = plsc.VectorSubcoreMesh(core_axis_name="c", subcore_axis_name="s",
                                  num_cores=NC, num_subcores=NS)
    out_type = [jax.ShapeDtypeStruct((NC * NP, D), jnp.float32)]
    scratch = [
        pltpu.VMEM((SUPER, C), jnp.int32),
        pltpu.VMEM((SUPER, C), jnp.int32),
        pltpu.VMEM((C, D), jnp.float32),
        pltpu.VMEM((C, 2 * D), jnp.float32),
        pltpu.VMEM_SHARED((NP, D), jnp.float32),
    ]
    if with_deg:
        out_type.append(jax.ShapeDtypeStruct((NC * NP, D), jnp.float32))

    def body(m_hbm, gb_hbm, src_hbm, dst_hbm, *refs):
        if with_deg:
            agg_out, deg_out = refs[0], refs[1]
            src_v, dst_v, mrow, gbrow, agg_sh = refs[2:]
        else:
            agg_out = refs[0]
            src_v, dst_v, mrow, gbrow, agg_sh = refs[1:]
        c = lax.axis_index("c")
        s = lax.axis_index("s")
        w = c * NS + s
        my_rows = s * ROWS_PS

        def fill_mrow(val):
            @pl.loop(0, C)
            def _fill(i):
                for j in range(D // L):
                    mrow[i, pl.ds(j * L, L)] = jnp.full((L,), val, jnp.float32)

        def zero_my_slice():
            @pl.loop(0, ROWS_PS // C)
            def _zero(k):
                pltpu.sync_copy(mrow, agg_sh.at[pl.ds(my_rows + k * C, C)])

        if stage >= 1:
            fill_mrow(0.0)
        if stage >= 2:
            zero_my_slice()
        if stage >= 3:
            plsc.subcore_barrier()

        if stage >= 5:
            @pl.loop(0, NSUPER)
            def _super(u):
                ibase = w * NCHUNK + u * SUPER
                pltpu.sync_copy(src_hbm.at[pl.ds(ibase, SUPER)], src_v)
                pltpu.sync_copy(dst_hbm.at[pl.ds(ibase, SUPER)], dst_v)

                @pl.loop(0, SUPER)
                def _chunk(t):
                    if stage >= 6:
                        pltpu.sync_copy(m_hbm.at[src_v.at[t]], mrow)
                        pltpu.sync_copy(gb_hbm.at[dst_v.at[t]], gbrow)

                    if stage >= 7:
                        @pl.loop(0, C)
                        def _edge(i):
                            for j in range(D // L):
                                off = j * L
                                mm = mrow[i, pl.ds(off, L)]
                                g = gbrow[i, pl.ds(D + off, L)]
                                b = gbrow[i, pl.ds(off, L)]
                                mrow[i, pl.ds(off, L)] = jnp.maximum(g * mm + b, 0.0)

                    if stage >= 8:
                        pltpu.sync_copy(mrow, agg_sh.at[dst_v.at[t]], add=True)

        if stage >= 3:
            plsc.subcore_barrier()

        if stage >= 4:
            pltpu.sync_copy(agg_sh.at[pl.ds(my_rows, ROWS_PS)],
                            agg_out.at[pl.ds(c * NP + my_rows, ROWS_PS)])

        if with_deg and stage >= 9:
            fill_mrow(0.0)
            zero_my_slice()
            fill_mrow(1.0)
            plsc.subcore_barrier()

            @pl.loop(0, NSUPER)
            def _dsuper(u):
                ibase = w * NCHUNK + u * SUPER
                pltpu.sync_copy(dst_hbm.at[pl.ds(ibase, SUPER)], dst_v)

                @pl.loop(0, SUPER)
                def _dchunk(t):
                    pltpu.sync_copy(mrow, agg_sh.at[dst_v.at[t]], add=True)

            plsc.subcore_barrier()
            pltpu.sync_copy(agg_sh.at[pl.ds(my_rows, ROWS_PS)],
                            deg_out.at[pl.ds(c * NP + my_rows, ROWS_PS)])

    return pl.kernel(body, out_type=out_type, mesh=mesh, scratch_types=scratch)


_STAGE = 9
_sc_edge_deg = _make_sc_edge(True, _STAGE)
_sc_edge = _make_sc_edge(False, _STAGE)


def kernel(x, edge_index, edge_weight, Wf1, bf1, Wfs1, bfs1, Wl1, Wls1,
           Wf2, bf2, Wfs2, bfs2, Wl2, Wls2, Wo, bo):
    del edge_weight
    xp = jnp.pad(x, ((0, NP - N), (0, 0)))
    src = jnp.full((EP,), NP - 1, jnp.int32).at[:E].set(edge_index[0])
    dst = jnp.full((EP,), NP - 1, jnp.int32).at[:E].set(edge_index[1])
    src = src.reshape(NW * NCHUNK, C)
    dst = dst.reshape(NW * NCHUNK, C)

    gb1, m1, skip1 = _dense(xp, Wf1, bf1.reshape(1, -1), Wfs1, bfs1.reshape(1, -1), Wl1, Wls1)
    agg1, degp = _sc_edge_deg(m1, gb1, src, dst)
    h1 = _combine1(skip1, agg1, agg1, degp, degp)

    gb2, m2, skip2 = _dense(h1, Wf2, bf2.reshape(1, -1), Wfs2, bfs2.reshape(1, -1), Wl2, Wls2)
    (agg2,) = _sc_edge(m2, gb2, src, dst)
    y = _combine2(skip2, agg2, agg2, degp, degp, Wo, bo.reshape(1, 1))
    return y[:N]


# speedup vs baseline: 4.2470x; 2.2375x over previous
import functools

import jax
import jax.numpy as jnp
from jax import lax
from jax.experimental import pallas as pl
from jax.experimental.pallas import tpu as pltpu
from jax.experimental.pallas import tpu_sc as plsc

N = 10000
NP = 10240
D = 128
E = 320000
NC, NS, L = 2, 16, 16
NW = NC * NS
EPW = 10240
EP = NW * EPW
C = 32
NCHUNK = EPW // C
ROWS_PS = NP // NS
BR = 1280
GRID = NP // BR


def _dense_body(x_ref, wf_ref, bf_ref, wfs_ref, bfs_ref, wl_ref, wls_ref,
                gb_ref, m_ref, skip_ref):
    x = x_ref[...]
    gb_ref[...] = jnp.dot(x, wf_ref[...], preferred_element_type=jnp.float32) + bf_ref[...]
    bgs = jnp.dot(x, wfs_ref[...], preferred_element_type=jnp.float32) + bfs_ref[...]
    m_ref[...] = jnp.dot(x, wl_ref[...], preferred_element_type=jnp.float32)
    pre = bgs[:, D:] * jnp.dot(x, wls_ref[...], preferred_element_type=jnp.float32) + bgs[:, :D]
    skip_ref[...] = jnp.maximum(pre, 0.0)


_dense = pl.pallas_call(
    _dense_body,
    grid=(GRID,),
    in_specs=[
        pl.BlockSpec((BR, D), lambda i: (i, 0)),
        pl.BlockSpec((D, 2 * D), lambda i: (0, 0)),
        pl.BlockSpec((1, 2 * D), lambda i: (0, 0)),
        pl.BlockSpec((D, 2 * D), lambda i: (0, 0)),
        pl.BlockSpec((1, 2 * D), lambda i: (0, 0)),
        pl.BlockSpec((D, D), lambda i: (0, 0)),
        pl.BlockSpec((D, D), lambda i: (0, 0)),
    ],
    out_specs=[
        pl.BlockSpec((BR, 2 * D), lambda i: (i, 0)),
        pl.BlockSpec((BR, D), lambda i: (i, 0)),
        pl.BlockSpec((BR, D), lambda i: (i, 0)),
    ],
    out_shape=[
        jax.ShapeDtypeStruct((NP, 2 * D), jnp.float32),
        jax.ShapeDtypeStruct((NP, D), jnp.float32),
        jax.ShapeDtypeStruct((NP, D), jnp.float32),
    ],
)


def _combine1_body(skip_ref, a0_ref, a1_ref, d0_ref, d1_ref, h_ref):
    deg = d0_ref[...][:, :1] + d1_ref[...][:, :1]
    scale = 1.0 / jnp.maximum(deg, 1.0)
    h = skip_ref[...] + (a0_ref[...] + a1_ref[...]) * scale
    h_ref[...] = jnp.where(h >= 0, h, 0.01 * h)


_combine1 = pl.pallas_call(
    _combine1_body,
    grid=(GRID,),
    in_specs=[
        pl.BlockSpec((BR, D), lambda i: (i, 0)),
        pl.BlockSpec((BR, D), lambda i: (i, 0)),
        pl.BlockSpec((BR, D), lambda i: (GRID + i, 0)),
        pl.BlockSpec((BR, D), lambda i: (i, 0)),
        pl.BlockSpec((BR, D), lambda i: (GRID + i, 0)),
    ],
    out_specs=pl.BlockSpec((BR, D), lambda i: (i, 0)),
    out_shape=jax.ShapeDtypeStruct((NP, D), jnp.float32),
)


def _combine2_body(skip_ref, a0_ref, a1_ref, d0_ref, d1_ref, wo_ref, bo_ref, y_ref):
    deg = d0_ref[...][:, :1] + d1_ref[...][:, :1]
    scale = 1.0 / jnp.maximum(deg, 1.0)
    h = skip_ref[...] + (a0_ref[...] + a1_ref[...]) * scale
    h = jnp.where(h >= 0, h, 0.01 * h)
    y_ref[...] = jnp.dot(h, wo_ref[...], preferred_element_type=jnp.float32) + bo_ref[...]


_combine2 = pl.pallas_call(
    _combine2_body,
    grid=(GRID,),
    in_specs=[
        pl.BlockSpec((BR, D), lambda i: (i, 0)),
        pl.BlockSpec((BR, D), lambda i: (i, 0)),
        pl.BlockSpec((BR, D), lambda i: (GRID + i, 0)),
        pl.BlockSpec((BR, D), lambda i: (i, 0)),
        pl.BlockSpec((BR, D), lambda i: (GRID + i, 0)),
        pl.BlockSpec((D, 1), lambda i: (0, 0)),
        pl.BlockSpec((1, 1), lambda i: (0, 0)),
    ],
    out_specs=pl.BlockSpec((BR, 1), lambda i: (i, 0)),
    out_shape=jax.ShapeDtypeStruct((NP, 1), jnp.float32),
)


SUPER = 16
NSUPER = NCHUNK // SUPER
NB = 2


def _make_sc_edge(with_deg: bool):
    mesh = plsc.VectorSubcoreMesh(core_axis_name="c", subcore_axis_name="s",
                                  num_cores=NC, num_subcores=NS)
    out_type = [jax.ShapeDtypeStruct((NC * NP, D), jnp.float32)]
    scratch = [
        pltpu.VMEM((SUPER, C), jnp.int32),
        pltpu.VMEM((SUPER, C), jnp.int32),
        pltpu.VMEM((SUPER, C), jnp.int32),
        pltpu.VMEM((SUPER, C), jnp.int32),
        pltpu.VMEM((C, D), jnp.float32),
        pltpu.VMEM((C, D), jnp.float32),
        pltpu.VMEM((C, 2 * D), jnp.float32),
        pltpu.VMEM((C, 2 * D), jnp.float32),
        pltpu.VMEM((C, D), jnp.float32),
        pltpu.VMEM((C, D), jnp.float32),
        pltpu.VMEM_SHARED((NP, D), jnp.float32),
        pltpu.SemaphoreType.DMA,
        pltpu.SemaphoreType.DMA,
        pltpu.SemaphoreType.DMA,
        pltpu.SemaphoreType.DMA,
        pltpu.SemaphoreType.DMA,
        pltpu.SemaphoreType.DMA,
    ]
    if with_deg:
        out_type.append(jax.ShapeDtypeStruct((NC * NP, D), jnp.float32))

    def body(m_hbm, gb_hbm, src_hbm, dst_hbm, *refs):
        if with_deg:
            agg_out, deg_out = refs[0], refs[1]
            scr = refs[2:]
        else:
            agg_out = refs[0]
            scr = refs[1:]
        (src0, src1, dst0, dst1, mrow0, mrow1, gbrow0, gbrow1, msg0, msg1,
         agg_sh, sem_m0, sem_m1, sem_g0, sem_g1, sem_s0, sem_s1) = scr
        srcb, dstb = (src0, src1), (dst0, dst1)
        mrowb, gbrowb, msgb = (mrow0, mrow1), (gbrow0, gbrow1), (msg0, msg1)
        semm, semg, sems = (sem_m0, sem_m1), (sem_g0, sem_g1), (sem_s0, sem_s1)

        c = lax.axis_index("c")
        s = lax.axis_index("s")
        w = c * NS + s
        my_rows = s * ROWS_PS
        base_w = w * NCHUNK

        def fill(buf, val):
            @pl.loop(0, C)
            def _fill(i):
                for j in range(D // L):
                    buf[i, pl.ds(j * L, L)] = jnp.full((L,), val, jnp.float32)

        def zero_my_slice():
            @pl.loop(0, ROWS_PS // C)
            def _zero(k):
                pltpu.sync_copy(msg0, agg_sh.at[pl.ds(my_rows + k * C, C)])

        def wait_scat(b):
            pltpu.make_async_copy(msgb[b], agg_sh.at[dst0.at[0]], sems[b]).wait()

        def issue_gathers(b, q, k):
            pltpu.async_copy(m_hbm.at[srcb[q].at[k]], mrowb[b], semm[b])
            pltpu.async_copy(gb_hbm.at[dstb[q].at[k]], gbrowb[b], semg[b])

        def wait_gathers(b, q, k):
            pltpu.make_async_copy(m_hbm.at[srcb[q].at[k]], mrowb[b], semm[b]).wait()
            pltpu.make_async_copy(gb_hbm.at[dstb[q].at[k]], gbrowb[b], semg[b]).wait()

        fill(msg0, 0.0)
        zero_my_slice()
        plsc.subcore_barrier()

        pltpu.sync_copy(src_hbm.at[pl.ds(base_w, SUPER)], src0)
        pltpu.sync_copy(dst_hbm.at[pl.ds(base_w, SUPER)], dst0)
        issue_gathers(0, 0, 0)
        issue_gathers(1, 0, 1)

        @pl.loop(0, NSUPER // 2)
        def _upair(uu):
            for p in (0, 1):
                v = uu * 2 + p

                @pl.when(v + 1 < NSUPER)
                def _stage():
                    nxt = base_w + (v + 1) * SUPER
                    pltpu.sync_copy(src_hbm.at[pl.ds(nxt, SUPER)], srcb[1 - p])
                    pltpu.sync_copy(dst_hbm.at[pl.ds(nxt, SUPER)], dstb[1 - p])

                for k in range(SUPER):
                    b = k % 2

                    if k >= 2:
                        wait_scat(b)
                    else:
                        @pl.when(v >= 1)
                        def _ws():
                            wait_scat(b)

                    wait_gathers(b, p, k)

                    @pl.loop(0, C)
                    def _edge(i):
                        for j in range(D // L):
                            off = j * L
                            mm = mrowb[b][i, pl.ds(off, L)]
                            g = gbrowb[b][i, pl.ds(D + off, L)]
                            bb = gbrowb[b][i, pl.ds(off, L)]
                            msgb[b][i, pl.ds(off, L)] = jnp.maximum(g * mm + bb, 0.0)

                    if k < SUPER - 2:
                        issue_gathers(b, p, k + 2)
                    else:
                        @pl.when(v + 1 < NSUPER)
                        def _ig():
                            issue_gathers(b, 1 - p, k + 2 - SUPER)

                    pltpu.async_copy(msgb[b], agg_sh.at[dstb[p].at[k]],
                                     sems[b], add=True)

        wait_scat(0)
        wait_scat(1)
        plsc.subcore_barrier()

        pltpu.sync_copy(agg_sh.at[pl.ds(my_rows, ROWS_PS)],
                        agg_out.at[pl.ds(c * NP + my_rows, ROWS_PS)])

        if with_deg:
            fill(msg0, 0.0)
            zero_my_slice()
            fill(msg0, 1.0)
            plsc.subcore_barrier()

            @pl.loop(0, NSUPER)
            def _dsuper(u):
                pltpu.sync_copy(dst_hbm.at[pl.ds(base_w + u * SUPER, SUPER)], dst0)
                for k in range(SUPER):
                    pltpu.async_copy(msg0, agg_sh.at[dst0.at[k]], sem_s0, add=True)
                for k in range(SUPER):
                    pltpu.make_async_copy(msg0, agg_sh.at[dst0.at[0]], sem_s0).wait()

            plsc.subcore_barrier()
            pltpu.sync_copy(agg_sh.at[pl.ds(my_rows, ROWS_PS)],
                            deg_out.at[pl.ds(c * NP + my_rows, ROWS_PS)])

    return pl.kernel(body, out_type=out_type, mesh=mesh, scratch_types=scratch)


_sc_edge_deg = _make_sc_edge(True)
_sc_edge = _make_sc_edge(False)


def kernel(x, edge_index, edge_weight, Wf1, bf1, Wfs1, bfs1, Wl1, Wls1,
           Wf2, bf2, Wfs2, bfs2, Wl2, Wls2, Wo, bo):
    del edge_weight
    xp = jnp.pad(x, ((0, NP - N), (0, 0)))
    src = jnp.full((EP,), NP - 1, jnp.int32).at[:E].set(edge_index[0])
    dst = jnp.full((EP,), NP - 1, jnp.int32).at[:E].set(edge_index[1])
    src = src.reshape(NW * NCHUNK, C)
    dst = dst.reshape(NW * NCHUNK, C)

    gb1, m1, skip1 = _dense(xp, Wf1, bf1.reshape(1, -1), Wfs1, bfs1.reshape(1, -1), Wl1, Wls1)
    agg1, degp = _sc_edge_deg(m1, gb1, src, dst)
    h1 = _combine1(skip1, agg1, agg1, degp, degp)

    gb2, m2, skip2 = _dense(h1, Wf2, bf2.reshape(1, -1), Wfs2, bfs2.reshape(1, -1), Wl2, Wls2)
    (agg2,) = _sc_edge(m2, gb2, src, dst)
    y = _combine2(skip2, agg2, agg2, degp, degp, Wo, bo.reshape(1, 1))
    return y[:N]


# speedup vs baseline: 4.2633x; 1.0038x over previous
import functools

import jax
import jax.numpy as jnp
from jax import lax
from jax.experimental import pallas as pl
from jax.experimental.pallas import tpu as pltpu
from jax.experimental.pallas import tpu_sc as plsc

N = 10000
NP = 10240
D = 128
E = 320000
NC, NS, L = 2, 16, 16
NW = NC * NS
EPW = 10240
EP = NW * EPW
C = 32
NCHUNK = EPW // C
ROWS_PS = NP // NS
BR = 1280
GRID = NP // BR


def _dense_body(x_ref, wf_ref, bf_ref, wfs_ref, bfs_ref, wl_ref, wls_ref,
                gb_ref, m_ref, skip_ref):
    x = x_ref[...]
    gb_ref[...] = jnp.dot(x, wf_ref[...], preferred_element_type=jnp.float32) + bf_ref[...]
    bgs = jnp.dot(x, wfs_ref[...], preferred_element_type=jnp.float32) + bfs_ref[...]
    m_ref[...] = jnp.dot(x, wl_ref[...], preferred_element_type=jnp.float32)
    pre = bgs[:, D:] * jnp.dot(x, wls_ref[...], preferred_element_type=jnp.float32) + bgs[:, :D]
    skip_ref[...] = jnp.maximum(pre, 0.0)


_dense = pl.pallas_call(
    _dense_body,
    grid=(GRID,),
    in_specs=[
        pl.BlockSpec((BR, D), lambda i: (i, 0)),
        pl.BlockSpec((D, 2 * D), lambda i: (0, 0)),
        pl.BlockSpec((1, 2 * D), lambda i: (0, 0)),
        pl.BlockSpec((D, 2 * D), lambda i: (0, 0)),
        pl.BlockSpec((1, 2 * D), lambda i: (0, 0)),
        pl.BlockSpec((D, D), lambda i: (0, 0)),
        pl.BlockSpec((D, D), lambda i: (0, 0)),
    ],
    out_specs=[
        pl.BlockSpec((BR, 2 * D), lambda i: (i, 0)),
        pl.BlockSpec((BR, D), lambda i: (i, 0)),
        pl.BlockSpec((BR, D), lambda i: (i, 0)),
    ],
    out_shape=[
        jax.ShapeDtypeStruct((NP, 2 * D), jnp.float32),
        jax.ShapeDtypeStruct((NP, D), jnp.float32),
        jax.ShapeDtypeStruct((NP, D), jnp.float32),
    ],
)


def _combine1_body(skip_ref, a0_ref, a1_ref, d0_ref, d1_ref, h_ref):
    deg = d0_ref[...][:, :1] + d1_ref[...][:, :1]
    scale = 1.0 / jnp.maximum(deg, 1.0)
    h = skip_ref[...] + (a0_ref[...] + a1_ref[...]) * scale
    h_ref[...] = jnp.where(h >= 0, h, 0.01 * h)


_combine1 = pl.pallas_call(
    _combine1_body,
    grid=(GRID,),
    in_specs=[
        pl.BlockSpec((BR, D), lambda i: (i, 0)),
        pl.BlockSpec((BR, D), lambda i: (i, 0)),
        pl.BlockSpec((BR, D), lambda i: (GRID + i, 0)),
        pl.BlockSpec((BR, D), lambda i: (i, 0)),
        pl.BlockSpec((BR, D), lambda i: (GRID + i, 0)),
    ],
    out_specs=pl.BlockSpec((BR, D), lambda i: (i, 0)),
    out_shape=jax.ShapeDtypeStruct((NP, D), jnp.float32),
)


def _combine2_body(skip_ref, a0_ref, a1_ref, d0_ref, d1_ref, wo_ref, bo_ref, y_ref):
    deg = d0_ref[...][:, :1] + d1_ref[...][:, :1]
    scale = 1.0 / jnp.maximum(deg, 1.0)
    h = skip_ref[...] + (a0_ref[...] + a1_ref[...]) * scale
    h = jnp.where(h >= 0, h, 0.01 * h)
    y_ref[...] = jnp.dot(h, wo_ref[...], preferred_element_type=jnp.float32) + bo_ref[...]


_combine2 = pl.pallas_call(
    _combine2_body,
    grid=(GRID,),
    in_specs=[
        pl.BlockSpec((BR, D), lambda i: (i, 0)),
        pl.BlockSpec((BR, D), lambda i: (i, 0)),
        pl.BlockSpec((BR, D), lambda i: (GRID + i, 0)),
        pl.BlockSpec((BR, D), lambda i: (i, 0)),
        pl.BlockSpec((BR, D), lambda i: (GRID + i, 0)),
        pl.BlockSpec((D, 1), lambda i: (0, 0)),
        pl.BlockSpec((1, 1), lambda i: (0, 0)),
    ],
    out_specs=pl.BlockSpec((BR, 1), lambda i: (i, 0)),
    out_shape=jax.ShapeDtypeStruct((NP, 1), jnp.float32),
)


SUPER = 16
NSUPER = NCHUNK // SUPER
NB = 2


def _make_sc_edge(with_deg: bool):
    mesh = plsc.VectorSubcoreMesh(core_axis_name="c", subcore_axis_name="s",
                                  num_cores=NC, num_subcores=NS)
    out_type = [jax.ShapeDtypeStruct((NC * NP, D), jnp.float32)]
    scratch = [
        pltpu.VMEM((SUPER, C), jnp.int32),
        pltpu.VMEM((SUPER, C), jnp.int32),
        pltpu.VMEM((SUPER, C), jnp.int32),
        pltpu.VMEM((SUPER, C), jnp.int32),
        pltpu.VMEM((C, D), jnp.float32),
        pltpu.VMEM((C, D), jnp.float32),
        pltpu.VMEM((C, 2 * D), jnp.float32),
        pltpu.VMEM((C, 2 * D), jnp.float32),
        pltpu.VMEM((C, D), jnp.float32),
        pltpu.VMEM((C, D), jnp.float32),
        pltpu.VMEM_SHARED((NP, D), jnp.float32),
        pltpu.SemaphoreType.DMA,
        pltpu.SemaphoreType.DMA,
        pltpu.SemaphoreType.DMA,
        pltpu.SemaphoreType.DMA,
        pltpu.SemaphoreType.DMA,
        pltpu.SemaphoreType.DMA,
    ]
    if with_deg:
        out_type.append(jax.ShapeDtypeStruct((NC * NP, D), jnp.float32))

    def body(m_hbm, gb_hbm, src_hbm, dst_hbm, *refs):
        if with_deg:
            agg_out, deg_out = refs[0], refs[1]
            scr = refs[2:]
        else:
            agg_out = refs[0]
            scr = refs[1:]
        (src0, src1, dst0, dst1, mrow0, mrow1, gbrow0, gbrow1, msg0, msg1,
         agg_sh, sem_m0, sem_m1, sem_g0, sem_g1, sem_s0, sem_s1) = scr
        srcb, dstb = (src0, src1), (dst0, dst1)
        mrowb, gbrowb, msgb = (mrow0, mrow1), (gbrow0, gbrow1), (msg0, msg1)
        semm, semg, sems = (sem_m0, sem_m1), (sem_g0, sem_g1), (sem_s0, sem_s1)

        c = lax.axis_index("c")
        s = lax.axis_index("s")
        w = c * NS + s
        my_rows = s * ROWS_PS
        base_w = w * NCHUNK

        def fill(buf, val):
            @pl.loop(0, C)
            def _fill(i):
                for j in range(D // L):
                    buf[i, pl.ds(j * L, L)] = jnp.full((L,), val, jnp.float32)

        def zero_my_slice():
            @pl.loop(0, ROWS_PS // C)
            def _zero(k):
                pltpu.sync_copy(msg0, agg_sh.at[pl.ds(my_rows + k * C, C)])

        def wait_scat(b):
            pltpu.make_async_copy(msgb[b], agg_sh.at[dst0.at[0]], sems[b]).wait()

        def issue_gathers(b, q, k):
            pltpu.async_copy(m_hbm.at[srcb[q].at[k]], mrowb[b], semm[b])
            pltpu.async_copy(gb_hbm.at[dstb[q].at[k]], gbrowb[b], semg[b])

        def wait_gathers(b, q, k):
            pltpu.make_async_copy(m_hbm.at[srcb[q].at[k]], mrowb[b], semm[b]).wait()
            pltpu.make_async_copy(gb_hbm.at[dstb[q].at[k]], gbrowb[b], semg[b]).wait()

        fill(msg0, 0.0)
        zero_my_slice()
        plsc.subcore_barrier()

        pltpu.sync_copy(src_hbm.at[pl.ds(base_w, SUPER)], src0)
        pltpu.sync_copy(dst_hbm.at[pl.ds(base_w, SUPER)], dst0)
        issue_gathers(0, 0, 0)
        issue_gathers(1, 0, 1)

        @pl.loop(0, NSUPER // 2)
        def _upair(uu):
            for p in (0, 1):
                v = uu * 2 + p

                @pl.when(v + 1 < NSUPER)
                def _stage():
                    nxt = base_w + (v + 1) * SUPER
                    pltpu.sync_copy(src_hbm.at[pl.ds(nxt, SUPER)], srcb[1 - p])
                    pltpu.sync_copy(dst_hbm.at[pl.ds(nxt, SUPER)], dstb[1 - p])

                @pl.loop(0, SUPER // 2)
                def _kpair(kk):
                    for b in (0, 1):
                        k = kk * 2 + b

                        @pl.when(jnp.logical_or(v >= 1, k >= 2))
                        def _ws():
                            wait_scat(b)

                        wait_gathers(b, p, k)

                        @pl.loop(0, C, unroll=4)
                        def _edge(i):
                            for j in range(D // L):
                                off = j * L
                                mm = mrowb[b][i, pl.ds(off, L)]
                                g = gbrowb[b][i, pl.ds(D + off, L)]
                                bb = gbrowb[b][i, pl.ds(off, L)]
                                msgb[b][i, pl.ds(off, L)] = jnp.maximum(g * mm + bb, 0.0)

                        @pl.when(k < SUPER - 2)
                        def _ig0():
                            issue_gathers(b, p, k + 2)

                        @pl.when(jnp.logical_and(k >= SUPER - 2, v + 1 < NSUPER))
                        def _ig1():
                            issue_gathers(b, 1 - p, k - (SUPER - 2))

                        pltpu.async_copy(msgb[b], agg_sh.at[dstb[p].at[k]],
                                         sems[b], add=True)

        wait_scat(0)
        wait_scat(1)
        plsc.subcore_barrier()

        pltpu.sync_copy(agg_sh.at[pl.ds(my_rows, ROWS_PS)],
                        agg_out.at[pl.ds(c * NP + my_rows, ROWS_PS)])

        if with_deg:
            fill(msg0, 0.0)
            zero_my_slice()
            fill(msg0, 1.0)
            plsc.subcore_barrier()

            @pl.loop(0, NSUPER)
            def _dsuper(u):
                pltpu.sync_copy(dst_hbm.at[pl.ds(base_w + u * SUPER, SUPER)], dst0)
                for k in range(SUPER):
                    pltpu.async_copy(msg0, agg_sh.at[dst0.at[k]], sem_s0, add=True)
                for k in range(SUPER):
                    pltpu.make_async_copy(msg0, agg_sh.at[dst0.at[0]], sem_s0).wait()

            plsc.subcore_barrier()
            pltpu.sync_copy(agg_sh.at[pl.ds(my_rows, ROWS_PS)],
                            deg_out.at[pl.ds(c * NP + my_rows, ROWS_PS)])

    return pl.kernel(body, out_type=out_type, mesh=mesh, scratch_types=scratch)


_sc_edge_deg = _make_sc_edge(True)
_sc_edge = _make_sc_edge(False)


def kernel(x, edge_index, edge_weight, Wf1, bf1, Wfs1, bfs1, Wl1, Wls1,
           Wf2, bf2, Wfs2, bfs2, Wl2, Wls2, Wo, bo):
    del edge_weight
    xp = jnp.pad(x, ((0, NP - N), (0, 0)))
    src = jnp.full((EP,), NP - 1, jnp.int32).at[:E].set(edge_index[0])
    dst = jnp.full((EP,), NP - 1, jnp.int32).at[:E].set(edge_index[1])
    src = src.reshape(NW * NCHUNK, C)
    dst = dst.reshape(NW * NCHUNK, C)

    gb1, m1, skip1 = _dense(xp, Wf1, bf1.reshape(1, -1), Wfs1, bfs1.reshape(1, -1), Wl1, Wls1)
    agg1, degp = _sc_edge_deg(m1, gb1, src, dst)
    h1 = _combine1(skip1, agg1, agg1, degp, degp)

    gb2, m2, skip2 = _dense(h1, Wf2, bf2.reshape(1, -1), Wfs2, bfs2.reshape(1, -1), Wl2, Wls2)
    (agg2,) = _sc_edge(m2, gb2, src, dst)
    y = _combine2(skip2, agg2, agg2, degp, degp, Wo, bo.reshape(1, 1))
    return y[:N]


# speedup vs baseline: 4.4666x; 1.0477x over previous
import dataclasses
import functools

import jax
import jax.numpy as jnp
import numpy as np
from jax import lax
from jax.experimental import pallas as pl
from jax.experimental.pallas import tpu as pltpu
from jax.experimental.pallas import tpu_sc as plsc

N = 10000
NP = 10240
D = 128
E = 320000
NC, NS, L = 2, 16, 16
NW = NC * NS
EPW = 10240
EP = NW * EPW
C = 32
NCHUNK = EPW // C
ROWS_PS = NP // NS
BR = 1280
GRID = NP // BR


def _dense_body(x_ref, wf_ref, bf_ref, wfs_ref, bfs_ref, wl_ref, wls_ref,
                gb_ref, m_ref, skip_ref):
    x = x_ref[...]
    gb = jnp.dot(x, wf_ref[...], preferred_element_type=jnp.float32) + bf_ref[...]
    gb_ref[...] = gb.astype(jnp.bfloat16)
    bgs = jnp.dot(x, wfs_ref[...], preferred_element_type=jnp.float32) + bfs_ref[...]
    m_ref[...] = jnp.dot(x, wl_ref[...], preferred_element_type=jnp.float32)
    pre = bgs[:, D:] * jnp.dot(x, wls_ref[...], preferred_element_type=jnp.float32) + bgs[:, :D]
    skip_ref[...] = jnp.maximum(pre, 0.0)


_dense = pl.pallas_call(
    _dense_body,
    grid=(GRID,),
    in_specs=[
        pl.BlockSpec((BR, D), lambda i: (i, 0)),
        pl.BlockSpec((D, 2 * D), lambda i: (0, 0)),
        pl.BlockSpec((1, 2 * D), lambda i: (0, 0)),
        pl.BlockSpec((D, 2 * D), lambda i: (0, 0)),
        pl.BlockSpec((1, 2 * D), lambda i: (0, 0)),
        pl.BlockSpec((D, D), lambda i: (0, 0)),
        pl.BlockSpec((D, D), lambda i: (0, 0)),
    ],
    out_specs=[
        pl.BlockSpec((BR, 2 * D), lambda i: (i, 0)),
        pl.BlockSpec((BR, D), lambda i: (i, 0)),
        pl.BlockSpec((BR, D), lambda i: (i, 0)),
    ],
    out_shape=[
        jax.ShapeDtypeStruct((NP, 2 * D), jnp.bfloat16),
        jax.ShapeDtypeStruct((NP, D), jnp.float32),
        jax.ShapeDtypeStruct((NP, D), jnp.float32),
    ],
)

_PERM128 = np.empty((D,), np.int32)
for _g in range(D // 32):
    for _i in range(16):
        _PERM128[32 * _g + 2 * _i] = 32 * _g + _i
        _PERM128[32 * _g + 2 * _i + 1] = 32 * _g + 16 + _i
_PERM256 = np.concatenate([_PERM128, D + _PERM128])


def _combine1_body(skip_ref, a0_ref, a1_ref, d0_ref, d1_ref, h_ref):
    deg = d0_ref[...][:, :1] + d1_ref[...][:, :1]
    scale = 1.0 / jnp.maximum(deg, 1.0)
    h = skip_ref[...] + (a0_ref[...] + a1_ref[...]) * scale
    h_ref[...] = jnp.where(h >= 0, h, 0.01 * h)


_combine1 = pl.pallas_call(
    _combine1_body,
    grid=(GRID,),
    in_specs=[
        pl.BlockSpec((BR, D), lambda i: (i, 0)),
        pl.BlockSpec((BR, D), lambda i: (i, 0)),
        pl.BlockSpec((BR, D), lambda i: (GRID + i, 0)),
        pl.BlockSpec((BR, D), lambda i: (i, 0)),
        pl.BlockSpec((BR, D), lambda i: (GRID + i, 0)),
    ],
    out_specs=pl.BlockSpec((BR, D), lambda i: (i, 0)),
    out_shape=jax.ShapeDtypeStruct((NP, D), jnp.float32),
)


def _combine2_body(skip_ref, a0_ref, a1_ref, d0_ref, d1_ref, wo_ref, bo_ref, y_ref):
    deg = d0_ref[...][:, :1] + d1_ref[...][:, :1]
    scale = 1.0 / jnp.maximum(deg, 1.0)
    h = skip_ref[...] + (a0_ref[...] + a1_ref[...]) * scale
    h = jnp.where(h >= 0, h, 0.01 * h)
    y_ref[...] = jnp.dot(h, wo_ref[...], preferred_element_type=jnp.float32) + bo_ref[...]


_combine2 = pl.pallas_call(
    _combine2_body,
    grid=(GRID,),
    in_specs=[
        pl.BlockSpec((BR, D), lambda i: (i, 0)),
        pl.BlockSpec((BR, D), lambda i: (i, 0)),
        pl.BlockSpec((BR, D), lambda i: (GRID + i, 0)),
        pl.BlockSpec((BR, D), lambda i: (i, 0)),
        pl.BlockSpec((BR, D), lambda i: (GRID + i, 0)),
        pl.BlockSpec((D, 1), lambda i: (0, 0)),
        pl.BlockSpec((1, 1), lambda i: (0, 0)),
    ],
    out_specs=pl.BlockSpec((BR, 1), lambda i: (i, 0)),
    out_shape=jax.ShapeDtypeStruct((NP, 1), jnp.float32),
)


SUPER = 16
NSUPER = NCHUNK // SUPER
NB = 2


def _make_sc_edge(with_deg: bool):
    mesh = plsc.VectorSubcoreMesh(core_axis_name="c", subcore_axis_name="s",
                                  num_cores=NC, num_subcores=NS)
    out_type = [jax.ShapeDtypeStruct((NC * NP, D), jnp.float32)]
    scratch = [
        pltpu.VMEM((SUPER, C), jnp.int32),
        pltpu.VMEM((SUPER, C), jnp.int32),
        pltpu.VMEM((SUPER, C), jnp.int32),
        pltpu.VMEM((SUPER, C), jnp.int32),
        pltpu.VMEM((C, D), jnp.float32),
        pltpu.VMEM((C, D), jnp.float32),
        pltpu.VMEM((C, D), jnp.int32),
        pltpu.VMEM((C, D), jnp.int32),
        pltpu.VMEM((C, D), jnp.float32),
        pltpu.VMEM((C, D), jnp.float32),
        pltpu.VMEM_SHARED((NP, D), jnp.float32),
        pltpu.SemaphoreType.DMA,
        pltpu.SemaphoreType.DMA,
        pltpu.SemaphoreType.DMA,
        pltpu.SemaphoreType.DMA,
        pltpu.SemaphoreType.DMA,
        pltpu.SemaphoreType.DMA,
    ]
    if with_deg:
        out_type.append(jax.ShapeDtypeStruct((NC * NP, D), jnp.float32))

    def body(m_hbm, gb_hbm, src_hbm, dst_hbm, *refs):
        if with_deg:
            agg_out, deg_out = refs[0], refs[1]
            scr = refs[2:]
        else:
            agg_out = refs[0]
            scr = refs[1:]
        (src0, src1, dst0, dst1, mrow0, mrow1, gbrow0, gbrow1, msg0, msg1,
         agg_sh, sem_m0, sem_m1, sem_g0, sem_g1, sem_s0, sem_s1) = scr
        srcb, dstb = (src0, src1), (dst0, dst1)
        mrowb, gbrowb, msgb = (mrow0, mrow1), (gbrow0, gbrow1), (msg0, msg1)
        semm, semg, sems = (sem_m0, sem_m1), (sem_g0, sem_g1), (sem_s0, sem_s1)

        c = lax.axis_index("c")
        s = lax.axis_index("s")
        w = c * NS + s
        my_rows = s * ROWS_PS
        base_w = w * NCHUNK

        def fill(buf, val):
            @pl.loop(0, C)
            def _fill(i):
                for j in range(D // L):
                    buf[i, pl.ds(j * L, L)] = jnp.full((L,), val, jnp.float32)

        def zero_my_slice():
            @pl.loop(0, ROWS_PS // C)
            def _zero(k):
                pltpu.sync_copy(msg0, agg_sh.at[pl.ds(my_rows + k * C, C)])

        def wait_scat(b):
            pltpu.make_async_copy(msgb[b], agg_sh.at[dst0.at[0]], sems[b]).wait()

        def issue_gathers(b, q, k):
            pltpu.async_copy(m_hbm.at[srcb[q].at[k]], mrowb[b], semm[b])
            pltpu.async_copy(gb_hbm.at[dstb[q].at[k]], gbrowb[b], semg[b])

        def wait_gathers(b, q, k):
            pltpu.make_async_copy(m_hbm.at[srcb[q].at[k]], mrowb[b], semm[b]).wait()
            pltpu.make_async_copy(gb_hbm.at[dstb[q].at[k]], gbrowb[b], semg[b]).wait()

        fill(msg0, 0.0)
        zero_my_slice()
        plsc.subcore_barrier()

        pltpu.sync_copy(src_hbm.at[pl.ds(base_w, SUPER)], src0)
        pltpu.sync_copy(dst_hbm.at[pl.ds(base_w, SUPER)], dst0)
        issue_gathers(0, 0, 0)
        issue_gathers(1, 0, 1)

        @pl.loop(0, NSUPER // 2)
        def _upair(uu):
            for p in (0, 1):
                v = uu * 2 + p

                @pl.when(v + 1 < NSUPER)
                def _stage():
                    nxt = base_w + (v + 1) * SUPER
                    pltpu.sync_copy(src_hbm.at[pl.ds(nxt, SUPER)], srcb[1 - p])
                    pltpu.sync_copy(dst_hbm.at[pl.ds(nxt, SUPER)], dstb[1 - p])

                @pl.loop(0, SUPER // 2)
                def _kpair(kk):
                    for b in (0, 1):
                        k = kk * 2 + b

                        @pl.when(jnp.logical_or(v >= 1, k >= 2))
                        def _ws():
                            wait_scat(b)

                        wait_gathers(b, p, k)

                        @pl.loop(0, C, unroll=4)
                        def _edge(i):
                            for j in range(D // 32):
                                off = j * L
                                b0, b1 = plsc.unpack(
                                    plsc.bitcast(gbrowb[b][i, pl.ds(off, L)], jnp.bfloat16),
                                    format=plsc.PackFormat.INTERLEAVED)
                                g0, g1 = plsc.unpack(
                                    plsc.bitcast(gbrowb[b][i, pl.ds(D // 2 + off, L)], jnp.bfloat16),
                                    format=plsc.PackFormat.INTERLEAVED)
                                m0 = mrowb[b][i, pl.ds(2 * off, L)]
                                m1 = mrowb[b][i, pl.ds(2 * off + L, L)]
                                msgb[b][i, pl.ds(2 * off, L)] = jnp.maximum(g0 * m0 + b0, 0.0)
                                msgb[b][i, pl.ds(2 * off + L, L)] = jnp.maximum(g1 * m1 + b1, 0.0)

                        @pl.when(k < SUPER - 2)
                        def _ig0():
                            issue_gathers(b, p, k + 2)

                        @pl.when(jnp.logical_and(k >= SUPER - 2, v + 1 < NSUPER))
                        def _ig1():
                            issue_gathers(b, 1 - p, k - (SUPER - 2))

                        pltpu.async_copy(msgb[b], agg_sh.at[dstb[p].at[k]],
                                         sems[b], add=True)

        wait_scat(0)
        wait_scat(1)
        plsc.subcore_barrier()

        pltpu.sync_copy(agg_sh.at[pl.ds(my_rows, ROWS_PS)],
                        agg_out.at[pl.ds(c * NP + my_rows, ROWS_PS)])

        if with_deg:
            fill(msg0, 0.0)
            zero_my_slice()
            fill(msg0, 1.0)
            plsc.subcore_barrier()

            @pl.loop(0, NSUPER)
            def _dsuper(u):
                pltpu.sync_copy(dst_hbm.at[pl.ds(base_w + u * SUPER, SUPER)], dst0)
                for k in range(SUPER):
                    pltpu.async_copy(msg0, agg_sh.at[dst0.at[k]], sem_s0, add=True)
                for k in range(SUPER):
                    pltpu.make_async_copy(msg0, agg_sh.at[dst0.at[0]], sem_s0).wait()

            plsc.subcore_barrier()
            pltpu.sync_copy(agg_sh.at[pl.ds(my_rows, ROWS_PS)],
                            deg_out.at[pl.ds(c * NP + my_rows, ROWS_PS)])

    cp = pltpu.CompilerParams()
    if "needs_layout_passes" in pltpu.CompilerParams.__dataclass_fields__:
        cp = dataclasses.replace(cp, needs_layout_passes=False)
    return pl.kernel(body, out_type=out_type, mesh=mesh, scratch_types=scratch,
                     compiler_params=cp)


_sc_edge_deg = _make_sc_edge(True)
_sc_edge = _make_sc_edge(False)


def kernel(x, edge_index, edge_weight, Wf1, bf1, Wfs1, bfs1, Wl1, Wls1,
           Wf2, bf2, Wfs2, bfs2, Wl2, Wls2, Wo, bo):
    del edge_weight
    xp = jnp.pad(x, ((0, NP - N), (0, 0)))
    src = jnp.full((EP,), NP - 1, jnp.int32).at[:E].set(edge_index[0])
    dst = jnp.full((EP,), NP - 1, jnp.int32).at[:E].set(edge_index[1])
    src = src.reshape(NW * NCHUNK, C)
    dst = dst.reshape(NW * NCHUNK, C)

    gb1, m1, skip1 = _dense(xp, Wf1[:, _PERM256], bf1[_PERM256].reshape(1, -1),
                            Wfs1, bfs1.reshape(1, -1), Wl1, Wls1)
    gb1 = jax.lax.bitcast_convert_type(gb1.reshape(NP, D, 2), jnp.int32)
    agg1, degp = _sc_edge_deg(m1, gb1, src, dst)
    h1 = _combine1(skip1, agg1, agg1, degp, degp)

    gb2, m2, skip2 = _dense(h1, Wf2[:, _PERM256], bf2[_PERM256].reshape(1, -1),
                            Wfs2, bfs2.reshape(1, -1), Wl2, Wls2)
    gb2 = jax.lax.bitcast_convert_type(gb2.reshape(NP, D, 2), jnp.int32)
    (agg2,) = _sc_edge(m2, gb2, src, dst)
    y = _combine2(skip2, agg2, agg2, degp, degp, Wo, bo.reshape(1, 1))
    return y[:N]


# speedup vs baseline: 4.5219x; 1.0124x over previous
import dataclasses
import functools

import jax
import jax.numpy as jnp
import numpy as np
from jax import lax
from jax.experimental import pallas as pl
from jax.experimental.pallas import tpu as pltpu
from jax.experimental.pallas import tpu_sc as plsc

N = 10000
NP = 10240
D = 128
E = 320000
NC, NS, L = 2, 16, 16
NW = NC * NS
EPW = 10240
EP = NW * EPW
C = 32
NCHUNK = EPW // C
ROWS_PS = NP // NS
BR = 1280
GRID = NP // BR


def _dense_body(x_ref, wf_ref, bf_ref, wfs_ref, bfs_ref, wl_ref, wls_ref,
                gb_ref, m_ref, skip_ref):
    x = x_ref[...]
    gb = jnp.dot(x, wf_ref[...], preferred_element_type=jnp.float32) + bf_ref[...]
    gb_ref[...] = gb.astype(jnp.bfloat16)
    bgs = jnp.dot(x, wfs_ref[...], preferred_element_type=jnp.float32) + bfs_ref[...]
    m_ref[...] = jnp.dot(x, wl_ref[...], preferred_element_type=jnp.float32)
    pre = bgs[:, D:] * jnp.dot(x, wls_ref[...], preferred_element_type=jnp.float32) + bgs[:, :D]
    skip_ref[...] = jnp.maximum(pre, 0.0)


_dense = pl.pallas_call(
    _dense_body,
    grid=(GRID,),
    in_specs=[
        pl.BlockSpec((BR, D), lambda i: (i, 0)),
        pl.BlockSpec((D, 2 * D), lambda i: (0, 0)),
        pl.BlockSpec((1, 2 * D), lambda i: (0, 0)),
        pl.BlockSpec((D, 2 * D), lambda i: (0, 0)),
        pl.BlockSpec((1, 2 * D), lambda i: (0, 0)),
        pl.BlockSpec((D, D), lambda i: (0, 0)),
        pl.BlockSpec((D, D), lambda i: (0, 0)),
    ],
    out_specs=[
        pl.BlockSpec((BR, 2 * D), lambda i: (i, 0)),
        pl.BlockSpec((BR, D), lambda i: (i, 0)),
        pl.BlockSpec((BR, D), lambda i: (i, 0)),
    ],
    out_shape=[
        jax.ShapeDtypeStruct((NP, 2 * D), jnp.bfloat16),
        jax.ShapeDtypeStruct((NP, D), jnp.float32),
        jax.ShapeDtypeStruct((NP, D), jnp.float32),
    ],
)

_PERM128 = np.empty((D,), np.int32)
for _g in range(D // 32):
    for _i in range(16):
        _PERM128[32 * _g + 2 * _i] = 32 * _g + _i
        _PERM128[32 * _g + 2 * _i + 1] = 32 * _g + 16 + _i
_PERM256 = np.concatenate([_PERM128, D + _PERM128])


def _combine1_body(skip_ref, a0_ref, a1_ref, d0_ref, d1_ref, h_ref):
    deg = d0_ref[...][:, :1] + d1_ref[...][:, :1]
    scale = 1.0 / jnp.maximum(deg, 1.0)
    h = skip_ref[...] + (a0_ref[...] + a1_ref[...]) * scale
    h_ref[...] = jnp.where(h >= 0, h, 0.01 * h)


_combine1 = pl.pallas_call(
    _combine1_body,
    grid=(GRID,),
    in_specs=[
        pl.BlockSpec((BR, D), lambda i: (i, 0)),
        pl.BlockSpec((BR, D), lambda i: (i, 0)),
        pl.BlockSpec((BR, D), lambda i: (GRID + i, 0)),
        pl.BlockSpec((BR, D), lambda i: (i, 0)),
        pl.BlockSpec((BR, D), lambda i: (GRID + i, 0)),
    ],
    out_specs=pl.BlockSpec((BR, D), lambda i: (i, 0)),
    out_shape=jax.ShapeDtypeStruct((NP, D), jnp.float32),
)


def _combine2_body(skip_ref, a0_ref, a1_ref, d0_ref, d1_ref, wo_ref, bo_ref, y_ref):
    deg = d0_ref[...][:, :1] + d1_ref[...][:, :1]
    scale = 1.0 / jnp.maximum(deg, 1.0)
    h = skip_ref[...] + (a0_ref[...] + a1_ref[...]) * scale
    h = jnp.where(h >= 0, h, 0.01 * h)
    y_ref[...] = jnp.dot(h, wo_ref[...], preferred_element_type=jnp.float32) + bo_ref[...]


_combine2 = pl.pallas_call(
    _combine2_body,
    grid=(GRID,),
    in_specs=[
        pl.BlockSpec((BR, D), lambda i: (i, 0)),
        pl.BlockSpec((BR, D), lambda i: (i, 0)),
        pl.BlockSpec((BR, D), lambda i: (GRID + i, 0)),
        pl.BlockSpec((BR, D), lambda i: (i, 0)),
        pl.BlockSpec((BR, D), lambda i: (GRID + i, 0)),
        pl.BlockSpec((D, 1), lambda i: (0, 0)),
        pl.BlockSpec((1, 1), lambda i: (0, 0)),
    ],
    out_specs=pl.BlockSpec((BR, 1), lambda i: (i, 0)),
    out_shape=jax.ShapeDtypeStruct((NP, 1), jnp.float32),
)


SUPER = 16
NSUPER = NCHUNK // SUPER
NB = 2


def _make_sc_edge(with_deg: bool):
    mesh = plsc.VectorSubcoreMesh(core_axis_name="c", subcore_axis_name="s",
                                  num_cores=NC, num_subcores=NS)
    out_type = [jax.ShapeDtypeStruct((NC * NP, D), jnp.float32)]
    scratch = [
        pltpu.VMEM((SUPER, C), jnp.int32),
        pltpu.VMEM((SUPER, C), jnp.int32),
        pltpu.VMEM((SUPER, C), jnp.int32),
        pltpu.VMEM((SUPER, C), jnp.int32),
        pltpu.VMEM((C, D), jnp.float32),
        pltpu.VMEM((C, D), jnp.float32),
        pltpu.VMEM((C, D), jnp.int32),
        pltpu.VMEM((C, D), jnp.int32),
        pltpu.VMEM((C, D), jnp.float32),
        pltpu.VMEM((C, D), jnp.float32),
        pltpu.VMEM_SHARED((NP, D), jnp.float32),
        pltpu.SemaphoreType.DMA,
        pltpu.SemaphoreType.DMA,
        pltpu.SemaphoreType.DMA,
        pltpu.SemaphoreType.DMA,
        pltpu.SemaphoreType.DMA,
        pltpu.SemaphoreType.DMA,
        pltpu.SemaphoreType.DMA,
        pltpu.SemaphoreType.DMA,
    ]
    if with_deg:
        out_type.append(jax.ShapeDtypeStruct((NC * NP, D), jnp.float32))

    def body(m_hbm, gb_hbm, src_hbm, dst_hbm, *refs):
        if with_deg:
            agg_out, deg_out = refs[0], refs[1]
            scr = refs[2:]
        else:
            agg_out = refs[0]
            scr = refs[1:]
        (src0, src1, dst0, dst1, mrow0, mrow1, gbrow0, gbrow1, msg0, msg1,
         agg_sh, sem_m0, sem_m1, sem_g0, sem_g1, sem_s0, sem_s1,
         sem_is, sem_id) = scr
        srcb, dstb = (src0, src1), (dst0, dst1)
        mrowb, gbrowb, msgb = (mrow0, mrow1), (gbrow0, gbrow1), (msg0, msg1)
        semm, semg, sems = (sem_m0, sem_m1), (sem_g0, sem_g1), (sem_s0, sem_s1)

        c = lax.axis_index("c")
        s = lax.axis_index("s")
        w = c * NS + s
        my_rows = s * ROWS_PS
        base_w = w * NCHUNK

        def fill(buf, val):
            @pl.loop(0, C)
            def _fill(i):
                for j in range(D // L):
                    buf[i, pl.ds(j * L, L)] = jnp.full((L,), val, jnp.float32)

        def zero_my_slice():
            @pl.loop(0, ROWS_PS // C)
            def _zero(k):
                pltpu.sync_copy(msg0, agg_sh.at[pl.ds(my_rows + k * C, C)])

        def wait_scat(b):
            pltpu.make_async_copy(msgb[b], agg_sh.at[dst0.at[0]], sems[b]).wait()

        def issue_gathers(b, q, k):
            pltpu.async_copy(m_hbm.at[srcb[q].at[k]], mrowb[b], semm[b])
            pltpu.async_copy(gb_hbm.at[dstb[q].at[k]], gbrowb[b], semg[b])

        def wait_gathers(b, q, k):
            pltpu.make_async_copy(m_hbm.at[srcb[q].at[k]], mrowb[b], semm[b]).wait()
            pltpu.make_async_copy(gb_hbm.at[dstb[q].at[k]], gbrowb[b], semg[b]).wait()

        fill(msg0, 0.0)
        zero_my_slice()
        plsc.subcore_barrier()

        pltpu.sync_copy(src_hbm.at[pl.ds(base_w, SUPER)], src0)
        pltpu.sync_copy(dst_hbm.at[pl.ds(base_w, SUPER)], dst0)
        issue_gathers(0, 0, 0)
        issue_gathers(1, 0, 1)

        @pl.loop(0, NSUPER // 2)
        def _upair(uu):
            for p in (0, 1):
                v = uu * 2 + p

                @pl.when(v + 1 < NSUPER)
                def _stage():
                    nxt = base_w + (v + 1) * SUPER
                    pltpu.async_copy(src_hbm.at[pl.ds(nxt, SUPER)], srcb[1 - p], sem_is)
                    pltpu.async_copy(dst_hbm.at[pl.ds(nxt, SUPER)], dstb[1 - p], sem_id)

                @pl.loop(0, SUPER // 2)
                def _kpair(kk):
                    for b in (0, 1):
                        k = kk * 2 + b

                        @pl.when(jnp.logical_or(v >= 1, k >= 2))
                        def _ws():
                            wait_scat(b)

                        wait_gathers(b, p, k)

                        @pl.loop(0, C, unroll=4)
                        def _edge(i):
                            for j in range(D // 32):
                                off = j * L
                                b0, b1 = plsc.unpack(
                                    plsc.bitcast(gbrowb[b][i, pl.ds(off, L)], jnp.bfloat16),
                                    format=plsc.PackFormat.INTERLEAVED)
                                g0, g1 = plsc.unpack(
                                    plsc.bitcast(gbrowb[b][i, pl.ds(D // 2 + off, L)], jnp.bfloat16),
                                    format=plsc.PackFormat.INTERLEAVED)
                                m0 = mrowb[b][i, pl.ds(2 * off, L)]
                                m1 = mrowb[b][i, pl.ds(2 * off + L, L)]
                                msgb[b][i, pl.ds(2 * off, L)] = jnp.maximum(g0 * m0 + b0, 0.0)
                                msgb[b][i, pl.ds(2 * off + L, L)] = jnp.maximum(g1 * m1 + b1, 0.0)

                        @pl.when(k < SUPER - 2)
                        def _ig0():
                            issue_gathers(b, p, k + 2)

                        @pl.when(jnp.logical_and(k == SUPER - 2, v + 1 < NSUPER))
                        def _wi():
                            nxt = base_w + (v + 1) * SUPER
                            pltpu.make_async_copy(
                                src_hbm.at[pl.ds(nxt, SUPER)], srcb[1 - p], sem_is).wait()
                            pltpu.make_async_copy(
                                dst_hbm.at[pl.ds(nxt, SUPER)], dstb[1 - p], sem_id).wait()

                        @pl.when(jnp.logical_and(k >= SUPER - 2, v + 1 < NSUPER))
                        def _ig1():
                            issue_gathers(b, 1 - p, k - (SUPER - 2))

                        pltpu.async_copy(msgb[b], agg_sh.at[dstb[p].at[k]],
                                         sems[b], add=True)

        wait_scat(0)
        wait_scat(1)
        plsc.subcore_barrier()

        pltpu.sync_copy(agg_sh.at[pl.ds(my_rows, ROWS_PS)],
                        agg_out.at[pl.ds(c * NP + my_rows, ROWS_PS)])

        if with_deg:
            fill(msg0, 0.0)
            zero_my_slice()
            fill(msg0, 1.0)
            plsc.subcore_barrier()

            @pl.loop(0, NSUPER)
            def _dsuper(u):
                pltpu.sync_copy(dst_hbm.at[pl.ds(base_w + u * SUPER, SUPER)], dst0)
                for k in range(SUPER):
                    pltpu.async_copy(msg0, agg_sh.at[dst0.at[k]], sem_s0, add=True)
                for k in range(SUPER):
                    pltpu.make_async_copy(msg0, agg_sh.at[dst0.at[0]], sem_s0).wait()

            plsc.subcore_barrier()
            pltpu.sync_copy(agg_sh.at[pl.ds(my_rows, ROWS_PS)],
                            deg_out.at[pl.ds(c * NP + my_rows, ROWS_PS)])

    cp = pltpu.CompilerParams()
    if "needs_layout_passes" in pltpu.CompilerParams.__dataclass_fields__:
        cp = dataclasses.replace(cp, needs_layout_passes=False)
    return pl.kernel(body, out_type=out_type, mesh=mesh, scratch_types=scratch,
                     compiler_params=cp)


_sc_edge_deg = _make_sc_edge(True)
_sc_edge = _make_sc_edge(False)


def kernel(x, edge_index, edge_weight, Wf1, bf1, Wfs1, bfs1, Wl1, Wls1,
           Wf2, bf2, Wfs2, bfs2, Wl2, Wls2, Wo, bo):
    del edge_weight
    xp = jnp.pad(x, ((0, NP - N), (0, 0)))
    src = jnp.full((EP,), NP - 1, jnp.int32).at[:E].set(edge_index[0])
    dst = jnp.full((EP,), NP - 1, jnp.int32).at[:E].set(edge_index[1])
    src = src.reshape(NW * NCHUNK, C)
    dst = dst.reshape(NW * NCHUNK, C)

    gb1, m1, skip1 = _dense(xp, Wf1[:, _PERM256], bf1[_PERM256].reshape(1, -1),
                            Wfs1, bfs1.reshape(1, -1), Wl1, Wls1)
    gb1 = jax.lax.bitcast_convert_type(gb1.reshape(NP, D, 2), jnp.int32)
    agg1, degp = _sc_edge_deg(m1, gb1, src, dst)
    h1 = _combine1(skip1, agg1, agg1, degp, degp)

    gb2, m2, skip2 = _dense(h1, Wf2[:, _PERM256], bf2[_PERM256].reshape(1, -1),
                            Wfs2, bfs2.reshape(1, -1), Wl2, Wls2)
    gb2 = jax.lax.bitcast_convert_type(gb2.reshape(NP, D, 2), jnp.int32)
    (agg2,) = _sc_edge(m2, gb2, src, dst)
    y = _combine2(skip2, agg2, agg2, degp, degp, Wo, bo.reshape(1, 1))
    return y[:N]


# speedup vs baseline: 4.6659x; 1.0319x over previous
import dataclasses
import functools

import jax
import jax.numpy as jnp
import numpy as np
from jax import lax
from jax.experimental import pallas as pl
from jax.experimental.pallas import tpu as pltpu
from jax.experimental.pallas import tpu_sc as plsc

N = 10000
NP = 10240
D = 128
E = 320000
NC, NS, L = 2, 16, 16
NW = NC * NS
EPW = 10240
EP = NW * EPW
C = 32
NCHUNK = EPW // C
ROWS_PS = NP // NS
BR = 1280
GRID = NP // BR


def _dense_body(x_ref, wf_ref, bf_ref, wfs_ref, bfs_ref, wl_ref, wls_ref,
                gb_ref, m_ref, skip_ref):
    x = x_ref[...]
    gb = jnp.dot(x, wf_ref[...], preferred_element_type=jnp.float32) + bf_ref[...]
    gb_ref[...] = gb.astype(jnp.bfloat16)
    bgs = jnp.dot(x, wfs_ref[...], preferred_element_type=jnp.float32) + bfs_ref[...]
    m_ref[...] = jnp.dot(x, wl_ref[...], preferred_element_type=jnp.float32)
    pre = bgs[:, D:] * jnp.dot(x, wls_ref[...], preferred_element_type=jnp.float32) + bgs[:, :D]
    skip_ref[...] = jnp.maximum(pre, 0.0)


_dense = pl.pallas_call(
    _dense_body,
    grid=(GRID,),
    in_specs=[
        pl.BlockSpec((BR, D), lambda i: (i, 0)),
        pl.BlockSpec((D, 2 * D), lambda i: (0, 0)),
        pl.BlockSpec((1, 2 * D), lambda i: (0, 0)),
        pl.BlockSpec((D, 2 * D), lambda i: (0, 0)),
        pl.BlockSpec((1, 2 * D), lambda i: (0, 0)),
        pl.BlockSpec((D, D), lambda i: (0, 0)),
        pl.BlockSpec((D, D), lambda i: (0, 0)),
    ],
    out_specs=[
        pl.BlockSpec((BR, 2 * D), lambda i: (i, 0)),
        pl.BlockSpec((BR, D), lambda i: (i, 0)),
        pl.BlockSpec((BR, D), lambda i: (i, 0)),
    ],
    out_shape=[
        jax.ShapeDtypeStruct((NP, 2 * D), jnp.bfloat16),
        jax.ShapeDtypeStruct((NP, D), jnp.float32),
        jax.ShapeDtypeStruct((NP, D), jnp.float32),
    ],
)

_PERM128 = np.empty((D,), np.int32)
for _g in range(D // 32):
    for _i in range(16):
        _PERM128[32 * _g + 2 * _i] = 32 * _g + _i
        _PERM128[32 * _g + 2 * _i + 1] = 32 * _g + 16 + _i
_PERM256 = np.concatenate([_PERM128, D + _PERM128])


def _combine1_body(skip_ref, a0_ref, a1_ref, d0_ref, d1_ref, h_ref):
    deg = d0_ref[...][:, :1] + d1_ref[...][:, :1]
    scale = 1.0 / jnp.maximum(deg, 1.0)
    h = skip_ref[...] + (a0_ref[...] + a1_ref[...]) * scale
    h_ref[...] = jnp.where(h >= 0, h, 0.01 * h)


_combine1 = pl.pallas_call(
    _combine1_body,
    grid=(GRID,),
    in_specs=[
        pl.BlockSpec((BR, D), lambda i: (i, 0)),
        pl.BlockSpec((BR, D), lambda i: (i, 0)),
        pl.BlockSpec((BR, D), lambda i: (GRID + i, 0)),
        pl.BlockSpec((BR, D), lambda i: (i, 0)),
        pl.BlockSpec((BR, D), lambda i: (GRID + i, 0)),
    ],
    out_specs=pl.BlockSpec((BR, D), lambda i: (i, 0)),
    out_shape=jax.ShapeDtypeStruct((NP, D), jnp.float32),
)


def _combine2_body(skip_ref, a0_ref, a1_ref, d0_ref, d1_ref, wo_ref, bo_ref, y_ref):
    deg = d0_ref[...][:, :1] + d1_ref[...][:, :1]
    scale = 1.0 / jnp.maximum(deg, 1.0)
    h = skip_ref[...] + (a0_ref[...] + a1_ref[...]) * scale
    h = jnp.where(h >= 0, h, 0.01 * h)
    y_ref[...] = jnp.dot(h, wo_ref[...], preferred_element_type=jnp.float32) + bo_ref[...]


_combine2 = pl.pallas_call(
    _combine2_body,
    grid=(GRID,),
    in_specs=[
        pl.BlockSpec((BR, D), lambda i: (i, 0)),
        pl.BlockSpec((BR, D), lambda i: (i, 0)),
        pl.BlockSpec((BR, D), lambda i: (GRID + i, 0)),
        pl.BlockSpec((BR, D), lambda i: (i, 0)),
        pl.BlockSpec((BR, D), lambda i: (GRID + i, 0)),
        pl.BlockSpec((D, 1), lambda i: (0, 0)),
        pl.BlockSpec((1, 1), lambda i: (0, 0)),
    ],
    out_specs=pl.BlockSpec((BR, 1), lambda i: (i, 0)),
    out_shape=jax.ShapeDtypeStruct((NP, 1), jnp.float32),
)


SUPER = 16
NSUPER = NCHUNK // SUPER
NB = 2


def _make_sc_edge(with_deg: bool):
    mesh = plsc.VectorSubcoreMesh(core_axis_name="c", subcore_axis_name="s",
                                  num_cores=NC, num_subcores=NS)
    out_type = [jax.ShapeDtypeStruct((NC * NP, D), jnp.float32)]
    scratch = [
        pltpu.VMEM((SUPER, C), jnp.int32),
        pltpu.VMEM((SUPER, C), jnp.int32),
        pltpu.VMEM((SUPER, C), jnp.int32),
        pltpu.VMEM((SUPER, C), jnp.int32),
        pltpu.VMEM((C, D), jnp.float32),
        pltpu.VMEM((C, D), jnp.float32),
        pltpu.VMEM((C, D), jnp.int32),
        pltpu.VMEM((C, D), jnp.int32),
        pltpu.VMEM((C, D), jnp.float32),
        pltpu.VMEM((C, D), jnp.float32),
        pltpu.VMEM_SHARED((NP, D), jnp.float32),
        pltpu.SemaphoreType.DMA,
        pltpu.SemaphoreType.DMA,
        pltpu.SemaphoreType.DMA,
        pltpu.SemaphoreType.DMA,
        pltpu.SemaphoreType.DMA,
        pltpu.SemaphoreType.DMA,
        pltpu.SemaphoreType.DMA,
        pltpu.SemaphoreType.DMA,
    ]
    if with_deg:
        out_type.append(jax.ShapeDtypeStruct((NC * NP, D), jnp.float32))

    def body(m_hbm, gb_hbm, src_hbm, dst_hbm, *refs):
        if with_deg:
            agg_out, deg_out = refs[0], refs[1]
            scr = refs[2:]
        else:
            agg_out = refs[0]
            scr = refs[1:]
        (src0, src1, dst0, dst1, mrow0, mrow1, gbrow0, gbrow1, msg0, msg1,
         agg_sh, sem_m0, sem_m1, sem_g0, sem_g1, sem_s0, sem_s1,
         sem_is, sem_id) = scr
        srcb, dstb = (src0, src1), (dst0, dst1)
        mrowb, gbrowb, msgb = (mrow0, mrow1), (gbrow0, gbrow1), (msg0, msg1)
        semm, semg, sems = (sem_m0, sem_m1), (sem_g0, sem_g1), (sem_s0, sem_s1)

        c = lax.axis_index("c")
        s = lax.axis_index("s")
        w = c * NS + s
        my_rows = s * ROWS_PS
        base_w = w * NCHUNK

        def fill(buf, val):
            @pl.loop(0, C)
            def _fill(i):
                for j in range(D // L):
                    buf[i, pl.ds(j * L, L)] = jnp.full((L,), val, jnp.float32)

        def zero_my_slice():
            @pl.loop(0, ROWS_PS // C)
            def _zero(k):
                pltpu.sync_copy(msg0, agg_sh.at[pl.ds(my_rows + k * C, C)])

        def wait_scat(b):
            pltpu.make_async_copy(msgb[b], agg_sh.at[dst0.at[0]], sems[b]).wait()

        def issue_gathers(b, q, k):
            pltpu.async_copy(m_hbm.at[srcb[q].at[k]], mrowb[b], semm[b])
            pltpu.async_copy(gb_hbm.at[dstb[q].at[k]], gbrowb[b], semg[b])

        def wait_gathers(b, q, k):
            pltpu.make_async_copy(m_hbm.at[srcb[q].at[k]], mrowb[b], semm[b]).wait()
            pltpu.make_async_copy(gb_hbm.at[dstb[q].at[k]], gbrowb[b], semg[b]).wait()

        fill(msg0, 0.0)
        zero_my_slice()
        plsc.subcore_barrier()

        pltpu.sync_copy(src_hbm.at[pl.ds(base_w, SUPER)], src0)
        pltpu.sync_copy(dst_hbm.at[pl.ds(base_w, SUPER)], dst0)
        issue_gathers(0, 0, 0)
        issue_gathers(1, 0, 1)

        @pl.loop(0, NSUPER // 2)
        def _upair(uu):
            for p in (0, 1):
                v = uu * 2 + p

                @pl.when(v + 1 < NSUPER)
                def _stage():
                    nxt = base_w + (v + 1) * SUPER
                    pltpu.async_copy(src_hbm.at[pl.ds(nxt, SUPER)], srcb[1 - p], sem_is)
                    pltpu.async_copy(dst_hbm.at[pl.ds(nxt, SUPER)], dstb[1 - p], sem_id)

                @pl.loop(0, SUPER // 2)
                def _kpair(kk):
                    for b in (0, 1):
                        k = kk * 2 + b

                        @pl.when(jnp.logical_or(v >= 1, k >= 2))
                        def _ws():
                            wait_scat(b)

                        wait_gathers(b, p, k)

                        @plsc.parallel_loop(0, C, unroll=4)
                        def _edge(i):
                            for j in range(D // 32):
                                off = j * L
                                b0, b1 = plsc.unpack(
                                    plsc.bitcast(gbrowb[b][i, pl.ds(off, L)], jnp.bfloat16),
                                    format=plsc.PackFormat.INTERLEAVED)
                                g0, g1 = plsc.unpack(
                                    plsc.bitcast(gbrowb[b][i, pl.ds(D // 2 + off, L)], jnp.bfloat16),
                                    format=plsc.PackFormat.INTERLEAVED)
                                m0 = mrowb[b][i, pl.ds(2 * off, L)]
                                m1 = mrowb[b][i, pl.ds(2 * off + L, L)]
                                msgb[b][i, pl.ds(2 * off, L)] = jnp.maximum(g0 * m0 + b0, 0.0)
                                msgb[b][i, pl.ds(2 * off + L, L)] = jnp.maximum(g1 * m1 + b1, 0.0)

                        @pl.when(k < SUPER - 2)
                        def _ig0():
                            issue_gathers(b, p, k + 2)

                        @pl.when(jnp.logical_and(k == SUPER - 2, v + 1 < NSUPER))
                        def _wi():
                            nxt = base_w + (v + 1) * SUPER
                            pltpu.make_async_copy(
                                src_hbm.at[pl.ds(nxt, SUPER)], srcb[1 - p], sem_is).wait()
                            pltpu.make_async_copy(
                                dst_hbm.at[pl.ds(nxt, SUPER)], dstb[1 - p], sem_id).wait()

                        @pl.when(jnp.logical_and(k >= SUPER - 2, v + 1 < NSUPER))
                        def _ig1():
                            issue_gathers(b, 1 - p, k - (SUPER - 2))

                        pltpu.async_copy(msgb[b], agg_sh.at[dstb[p].at[k]],
                                         sems[b], add=True)

        wait_scat(0)
        wait_scat(1)
        plsc.subcore_barrier()

        pltpu.sync_copy(agg_sh.at[pl.ds(my_rows, ROWS_PS)],
                        agg_out.at[pl.ds(c * NP + my_rows, ROWS_PS)])

        if with_deg:
            fill(msg0, 0.0)
            zero_my_slice()
            fill(msg0, 1.0)
            plsc.subcore_barrier()

            @pl.loop(0, NSUPER)
            def _dsuper(u):
                pltpu.sync_copy(dst_hbm.at[pl.ds(base_w + u * SUPER, SUPER)], dst0)
                for k in range(SUPER):
                    pltpu.async_copy(msg0, agg_sh.at[dst0.at[k]], sem_s0, add=True)
                for k in range(SUPER):
                    pltpu.make_async_copy(msg0, agg_sh.at[dst0.at[0]], sem_s0).wait()

            plsc.subcore_barrier()
            pltpu.sync_copy(agg_sh.at[pl.ds(my_rows, ROWS_PS)],
                            deg_out.at[pl.ds(c * NP + my_rows, ROWS_PS)])

    cp = pltpu.CompilerParams()
    if "needs_layout_passes" in pltpu.CompilerParams.__dataclass_fields__:
        cp = dataclasses.replace(cp, needs_layout_passes=False)
    return pl.kernel(body, out_type=out_type, mesh=mesh, scratch_types=scratch,
                     compiler_params=cp)


_sc_edge_deg = _make_sc_edge(True)
_sc_edge = _make_sc_edge(False)


def kernel(x, edge_index, edge_weight, Wf1, bf1, Wfs1, bfs1, Wl1, Wls1,
           Wf2, bf2, Wfs2, bfs2, Wl2, Wls2, Wo, bo):
    del edge_weight
    xp = jnp.pad(x, ((0, NP - N), (0, 0)))
    src = jnp.full((EP,), NP - 1, jnp.int32).at[:E].set(edge_index[0])
    dst = jnp.full((EP,), NP - 1, jnp.int32).at[:E].set(edge_index[1])
    src = src.reshape(NW * NCHUNK, C)
    dst = dst.reshape(NW * NCHUNK, C)

    gb1, m1, skip1 = _dense(xp, Wf1[:, _PERM256], bf1[_PERM256].reshape(1, -1),
                            Wfs1, bfs1.reshape(1, -1), Wl1, Wls1)
    gb1 = jax.lax.bitcast_convert_type(gb1.reshape(NP, D, 2), jnp.int32)
    agg1, degp = _sc_edge_deg(m1, gb1, src, dst)
    h1 = _combine1(skip1, agg1, agg1, degp, degp)

    gb2, m2, skip2 = _dense(h1, Wf2[:, _PERM256], bf2[_PERM256].reshape(1, -1),
                            Wfs2, bfs2.reshape(1, -1), Wl2, Wls2)
    gb2 = jax.lax.bitcast_convert_type(gb2.reshape(NP, D, 2), jnp.int32)
    (agg2,) = _sc_edge(m2, gb2, src, dst)
    y = _combine2(skip2, agg2, agg2, degp, degp, Wo, bo.reshape(1, 1))
    return y[:N]
